# baseline (speedup 1.0000x reference)
import jax
import jax.numpy as jnp
from jax.experimental import pallas as pl
from jax.experimental.pallas import tpu as pltpu


def _mlp_kernel(x_ref, w1a_ref, bv1_ref, w2a_ref, w3a_ref, o_ref):
    tB = x_ref.shape[1]
    h1 = jnp.dot(w1a_ref[...], x_ref[...], preferred_element_type=jnp.float32)
    h1 = jnp.maximum(h1 + bv1_ref[...], 0.0)

    h2 = jnp.dot(w2a_ref[...], h1, preferred_element_type=jnp.float32)
    h2 = jnp.maximum(h2, 0.0)

    y = jnp.dot(w3a_ref[...], h2, preferred_element_type=jnp.float32)
    o_ref[...] = y[0:1, :].reshape(tB // 128, 128)


def kernel(x_t, w1, b1, w2, b2, w3, b3):
    in_dim, B = x_t.shape
    hidden = w1.shape[0]

    w1a = jnp.zeros((16, in_dim), jnp.float32)
    w1a = w1a.at[:hidden, :].set(w1)

    bv1 = jnp.zeros((16, 1), jnp.float32)
    bv1 = bv1.at[:hidden, 0].set(b1)
    bv1 = bv1.at[hidden, 0].set(1.0)

    w2a = jnp.zeros((16, 16), jnp.float32)
    w2a = w2a.at[:hidden, :hidden].set(w2)
    w2a = w2a.at[:hidden, hidden].set(b2)
    w2a = w2a.at[hidden, hidden].set(1.0)

    w3a = jnp.zeros((8, 16), jnp.float32)
    w3a = w3a.at[0, :hidden].set(w3[0])
    w3a = w3a.at[0, hidden].set(b3[0])

    tB = 32768
    n_tiles = pl.cdiv(B, tB)
    B_pad = n_tiles * tB
    if B_pad != B:
        x_t = jnp.pad(x_t, ((0, 0), (0, B_pad - B)))

    out = pl.pallas_call(
        _mlp_kernel,
        out_shape=jax.ShapeDtypeStruct((B_pad // 128, 128), jnp.float32),
        grid=(n_tiles,),
        in_specs=[
            pl.BlockSpec((in_dim, tB), lambda i: (0, i)),
            pl.BlockSpec((16, in_dim), lambda i: (0, 0)),
            pl.BlockSpec((16, 1), lambda i: (0, 0)),
            pl.BlockSpec((16, 16), lambda i: (0, 0)),
            pl.BlockSpec((8, 16), lambda i: (0, 0)),
        ],
        out_specs=pl.BlockSpec((tB // 128, 128), lambda i: (i, 0)),
        compiler_params=pltpu.CompilerParams(
            dimension_semantics=("parallel",),
        ),
    )(x_t, w1a, bv1, w2a, w3a)

    return out.reshape(B_pad, 1)[:B]


# speedup vs baseline: 1.2497x; 1.2497x over previous
import jax
import jax.numpy as jnp
from jax.experimental import pallas as pl
from jax.experimental.pallas import tpu as pltpu


def _mlp_kernel(x_ref, w1a_ref, bv1_ref, w2a_ref, w3a_ref, o_ref):
    tB = x_ref.shape[1]
    h1 = jnp.dot(w1a_ref[...], x_ref[...], preferred_element_type=jnp.float32)
    h1 = jnp.maximum(h1 + bv1_ref[...], 0.0)

    h2 = jnp.dot(w2a_ref[...], h1, preferred_element_type=jnp.float32)
    h2 = jnp.maximum(h2, 0.0)

    y = jnp.dot(w3a_ref[...], h2, preferred_element_type=jnp.float32)
    o_ref[...] = y[0:1, :].reshape(tB // 128, 128)


def kernel(x_t, w1, b1, w2, b2, w3, b3):
    in_dim, B = x_t.shape
    hidden = w1.shape[0]

    w1a = jnp.zeros((16, in_dim), jnp.float32)
    w1a = w1a.at[:hidden, :].set(w1)

    bv1 = jnp.zeros((16, 1), jnp.float32)
    bv1 = bv1.at[:hidden, 0].set(b1)
    bv1 = bv1.at[hidden, 0].set(1.0)

    w2a = jnp.zeros((16, 16), jnp.float32)
    w2a = w2a.at[:hidden, :hidden].set(w2)
    w2a = w2a.at[:hidden, hidden].set(b2)
    w2a = w2a.at[hidden, hidden].set(1.0)

    w3a = jnp.zeros((8, 16), jnp.float32)
    w3a = w3a.at[0, :hidden].set(w3[0])
    w3a = w3a.at[0, hidden].set(b3[0])

    tB = 131072
    n_tiles = pl.cdiv(B, tB)
    B_pad = n_tiles * tB
    if B_pad != B:
        x_t = jnp.pad(x_t, ((0, 0), (0, B_pad - B)))

    out = pl.pallas_call(
        _mlp_kernel,
        out_shape=jax.ShapeDtypeStruct((B_pad // 128, 128), jnp.float32),
        grid=(n_tiles,),
        in_specs=[
            pl.BlockSpec((in_dim, tB), lambda i: (0, i)),
            pl.BlockSpec((16, in_dim), lambda i: (0, 0)),
            pl.BlockSpec((16, 1), lambda i: (0, 0)),
            pl.BlockSpec((16, 16), lambda i: (0, 0)),
            pl.BlockSpec((8, 16), lambda i: (0, 0)),
        ],
        out_specs=pl.BlockSpec((tB // 128, 128), lambda i: (i, 0)),
        compiler_params=pltpu.CompilerParams(
            dimension_semantics=("parallel",),
        ),
    )(x_t, w1a, bv1, w2a, w3a)

    return out.reshape(B_pad, 1)[:B]


# speedup vs baseline: 1.3800x; 1.1043x over previous
import jax
import jax.numpy as jnp
from jax.experimental import pallas as pl
from jax.experimental.pallas import tpu as pltpu


def _mlp_kernel(x_ref, wa_ref, o_ref):
    tB = x_ref.shape[1]
    w1a = wa_ref[0:16, 0:2]
    bv1 = wa_ref[0:16, 2:3]
    w2a = wa_ref[16:32, 0:16]
    w3a = wa_ref[32:40, 0:16]

    h1 = jnp.dot(w1a, x_ref[...], preferred_element_type=jnp.float32)
    h1 = jnp.maximum(h1 + bv1, 0.0)

    h2 = jnp.dot(w2a, h1, preferred_element_type=jnp.float32)
    h2 = jnp.maximum(h2, 0.0)

    y = jnp.dot(w3a, h2, preferred_element_type=jnp.float32)
    o_ref[...] = y[0:1, :].reshape(tB // 128, 128)


def kernel(x_t, w1, b1, w2, b2, w3, b3):
    in_dim, B = x_t.shape
    hidden = w1.shape[0]

    top = jnp.concatenate(
        [
            jnp.pad(w1, ((0, 16 - hidden), (0, 0))),
            jnp.pad(jnp.concatenate([b1, jnp.ones((1,), jnp.float32)]),
                    (0, 16 - hidden - 1))[:, None],
            jnp.zeros((16, 16 - in_dim - 1), jnp.float32),
        ],
        axis=1,
    )
    unit_row = jnp.zeros((1, 16), jnp.float32).at[0, hidden].set(1.0)
    mid = jnp.concatenate(
        [
            jnp.pad(jnp.concatenate([w2, b2[:, None]], axis=1),
                    ((0, 0), (0, 16 - hidden - 1))),
            unit_row,
            jnp.zeros((16 - hidden - 1, 16), jnp.float32),
        ],
        axis=0,
    )
    bot = jnp.pad(jnp.concatenate([w3, b3[:, None]], axis=1),
                  ((0, 7), (0, 16 - hidden - 1)))
    wa = jnp.concatenate([top, mid, bot], axis=0)

    tB = 262144
    n_tiles = pl.cdiv(B, tB)
    B_pad = n_tiles * tB
    if B_pad != B:
        x_t = jnp.pad(x_t, ((0, 0), (0, B_pad - B)))

    out = pl.pallas_call(
        _mlp_kernel,
        out_shape=jax.ShapeDtypeStruct((B_pad // 128, 128), jnp.float32),
        grid=(n_tiles,),
        in_specs=[
            pl.BlockSpec((in_dim, tB), lambda i: (0, i)),
            pl.BlockSpec((40, 16), lambda i: (0, 0)),
        ],
        out_specs=pl.BlockSpec((tB // 128, 128), lambda i: (i, 0)),
        compiler_params=pltpu.CompilerParams(
            dimension_semantics=("parallel",),
        ),
    )(x_t, wa)

    return out.reshape(B_pad, 1)[:B]


# speedup vs baseline: 1.3891x; 1.0066x over previous
import jax
import jax.numpy as jnp
from jax.experimental import pallas as pl
from jax.experimental.pallas import tpu as pltpu


def _mlp_kernel(x_ref, wa_ref, o_ref):
    tB = x_ref.shape[1]
    w1a = wa_ref[0:16, 0:2]
    bv1 = wa_ref[0:16, 2:3]
    w2a = wa_ref[16:32, 0:16]
    w3a = wa_ref[32:40, 0:16]

    h1 = jnp.dot(w1a, x_ref[...], preferred_element_type=jnp.float32)
    h1 = jnp.maximum(h1 + bv1, 0.0)

    h2 = jnp.dot(w2a, h1, preferred_element_type=jnp.float32)
    h2 = jnp.maximum(h2, 0.0)

    y = jnp.dot(w3a, h2, preferred_element_type=jnp.float32)
    o_ref[...] = y[0:1, :].reshape(tB // 128, 128)


def kernel(x_t, w1, b1, w2, b2, w3, b3):
    in_dim, B = x_t.shape
    hidden = w1.shape[0]

    top = jnp.concatenate(
        [
            jnp.pad(w1, ((0, 16 - hidden), (0, 0))),
            jnp.pad(jnp.concatenate([b1, jnp.ones((1,), jnp.float32)]),
                    (0, 16 - hidden - 1))[:, None],
            jnp.zeros((16, 16 - in_dim - 1), jnp.float32),
        ],
        axis=1,
    )
    unit_row = jnp.zeros((1, 16), jnp.float32).at[0, hidden].set(1.0)
    mid = jnp.concatenate(
        [
            jnp.pad(jnp.concatenate([w2, b2[:, None]], axis=1),
                    ((0, 0), (0, 16 - hidden - 1))),
            unit_row,
            jnp.zeros((16 - hidden - 1, 16), jnp.float32),
        ],
        axis=0,
    )
    bot = jnp.pad(jnp.concatenate([w3, b3[:, None]], axis=1),
                  ((0, 7), (0, 16 - hidden - 1)))
    wa = jnp.concatenate([top, mid, bot], axis=0)

    tB = 524288
    n_tiles = pl.cdiv(B, tB)
    B_pad = n_tiles * tB
    if B_pad != B:
        x_t = jnp.pad(x_t, ((0, 0), (0, B_pad - B)))

    out = pl.pallas_call(
        _mlp_kernel,
        out_shape=jax.ShapeDtypeStruct((B_pad // 128, 128), jnp.float32),
        grid=(n_tiles,),
        in_specs=[
            pl.BlockSpec((in_dim, tB), lambda i: (0, i)),
            pl.BlockSpec((40, 16), lambda i: (0, 0)),
        ],
        out_specs=pl.BlockSpec((tB // 128, 128), lambda i: (i, 0)),
        compiler_params=pltpu.CompilerParams(
            dimension_semantics=("parallel",),
        ),
    )(x_t, wa)

    return out.reshape(B_pad, 1)[:B]
